# baseline (speedup 1.0000x reference)
import jax
import jax.numpy as jnp
from jax import lax
from jax.experimental import pallas as pl
from jax.experimental.pallas import tpu as pltpu
from jax.experimental.pallas import tpu_sc as plsc

B, S, D = 4096, 200, 128
N = B * S
VOCAB = 100000
NC, NS = 2, 16
NW = NC * NS
TOK_PER_W = N // NW
K = 64
CHUNKS = TOK_PER_W // K
EPS = 1e-12

_mesh = plsc.VectorSubcoreMesh(
    core_axis_name="c", subcore_axis_name="s", num_cores=NC, num_subcores=NS)


def _bcast16(vec, j):
  return vec.at[jnp.full((16,), j, jnp.int32)].get(mode="promise_in_bounds")


def _sc_embed_ln_body(ids3_hbm, wemb_hbm, pos_hbm, semb_hbm,
                      g_hbm, b_hbm, out_hbm,
                      pos_tab, span_tab, g_tab, b_tab,
                      ibuf0, ibuf1, wbuf0, wbuf1, obuf0, obuf1, svb0, svb1,
                      isem0, isem1, gsem0, gsem1, osem0, osem1):
  wid = lax.axis_index("s") * NC + lax.axis_index("c")
  wbase = wid * TOK_PER_W
  cbase = wid * CHUNKS

  pltpu.sync_copy(pos_hbm, pos_tab)
  pltpu.sync_copy(semb_hbm, span_tab)
  pltpu.sync_copy(g_hbm, g_tab)
  pltpu.sync_copy(b_hbm, b_tab)

  lanes = lax.iota(jnp.int32, 16)
  zero = jnp.zeros((16,), jnp.float32)

  def issue_ids(c, ibuf, isem):
    pltpu.async_copy(ids3_hbm.at[cbase + c], ibuf, isem)

  def wait_ids(c, ibuf, isem):
    pltpu.make_async_copy(ids3_hbm.at[cbase + c], ibuf, isem).wait()

  def issue_gather(ibuf, wbuf, gsem):
    pltpu.async_copy(wemb_hbm.at[ibuf.at[0]], wbuf, gsem)

  def wait_gather(ibuf, wbuf, gsem):
    pltpu.make_async_copy(wemb_hbm.at[ibuf.at[0]], wbuf, gsem).wait()

  def issue_out(c, obuf, osem):
    pltpu.async_copy(obuf, out_hbm.at[pl.ds(wbase + c * K, K)], osem)

  def wait_out(c, obuf, osem):
    pltpu.make_async_copy(
        obuf, out_hbm.at[pl.ds(wbase + c * K, K)], osem).wait()

  def stage_sv(ibuf, svb):
    for g in range(K // 16):
      svb[pl.ds(g * 16, 16)] = ibuf[1, pl.ds(g * 16, 16)]

  def compute(cloc, svb, wbuf, obuf):
    gstrips = [g_tab[pl.ds(kk * 16, 16)] for kk in range(D // 16)]
    bstrips = [b_tab[pl.ds(kk * 16, 16)] for kk in range(D // 16)]

    def group_body(g, carry2):
      lt = g * 16 + lanes
      pv = lax.rem(cloc * K + lt, S)
      sv = svb[pl.ds(g * 16, 16)]

      def pass1(d, accs):
        s_, q_ = accs
        dl = lax.bitwise_and(lanes + d, D - 1)
        x = (plsc.load_gather(wbuf, [lt, dl])
             + plsc.load_gather(span_tab, [sv, dl])
             + plsc.load_gather(pos_tab, [pv, dl]))
        plsc.store_scatter(wbuf, [lt, dl], x)
        return s_ + x, q_ + x * x

      s_, q_ = plsc.parallel_loop(
          0, D, unroll=8, carry=(zero, zero))(pass1)
      mean = s_ * (1.0 / D)
      var = q_ * (1.0 / D) - mean * mean
      v = var + EPS
      y = plsc.bitcast(
          jnp.int32(0x5F3759DF) - lax.shift_right_logical(
              plsc.bitcast(v, jnp.int32), 1), jnp.float32)
      y = y * (1.5 - 0.5 * v * y * y)
      y = y * (1.5 - 0.5 * v * y * y)
      y = y * (1.5 - 0.5 * v * y * y)
      shift = -mean * y

      @plsc.parallel_loop(0, 16, unroll=4)
      def _(t):
        tok = g * 16 + t
        yt = _bcast16(y, t)
        st = _bcast16(shift, t)
        for kk in range(D // 16):
          x = wbuf[tok, pl.ds(kk * 16, 16)]
          obuf[tok, pl.ds(kk * 16, 16)] = (
              (x * yt + st) * gstrips[kk] + bstrips[kk])

      return carry2

    def two_groups(h, carry3):
      group_body(h * 2, 0)
      group_body(h * 2 + 1, 0)
      return carry3

    lax.fori_loop(0, K // 32, two_groups, 0)

  pltpu.sync_copy(ids3_hbm.at[cbase], ibuf0)
  issue_gather(ibuf0, wbuf0, gsem0)
  issue_ids(1, ibuf1, isem1)

  def step(i, carry):
    a = 2 * i
    b = a + 1
    wait_ids(b, ibuf1, isem1)
    issue_gather(ibuf1, wbuf1, gsem1)
    wait_gather(ibuf0, wbuf0, gsem0)
    stage_sv(ibuf0, svb0)

    @pl.when(a + 2 < CHUNKS)
    def _():
      issue_ids(a + 2, ibuf0, isem0)

    @pl.when(i > 0)
    def _():
      wait_out(a - 2, obuf0, osem0)

    compute(a, svb0, wbuf0, obuf0)
    issue_out(a, obuf0, osem0)

    @pl.when(a + 2 < CHUNKS)
    def _():
      wait_ids(a + 2, ibuf0, isem0)
      issue_gather(ibuf0, wbuf0, gsem0)

    wait_gather(ibuf1, wbuf1, gsem1)
    stage_sv(ibuf1, svb1)

    @pl.when(b + 2 < CHUNKS)
    def _():
      issue_ids(b + 2, ibuf1, isem1)

    @pl.when(i > 0)
    def _():
      wait_out(b - 2, obuf1, osem1)

    compute(b, svb1, wbuf1, obuf1)
    issue_out(b, obuf1, osem1)

    return carry

  lax.fori_loop(0, CHUNKS // 2, step, 0)
  wait_out(CHUNKS - 2, obuf0, osem0)
  wait_out(CHUNKS - 1, obuf1, osem1)


_sc_embed_ln = pl.kernel(
    _sc_embed_ln_body,
    out_type=jax.ShapeDtypeStruct((N, D), jnp.float32),
    mesh=_mesh,
    compiler_params=pltpu.CompilerParams(needs_layout_passes=False),
    scratch_types=[
        pltpu.VMEM((S, D), jnp.float32),
        pltpu.VMEM((512, D), jnp.float32),
        pltpu.VMEM((D,), jnp.float32),
        pltpu.VMEM((D,), jnp.float32),
        pltpu.VMEM((2, K), jnp.int32),
        pltpu.VMEM((2, K), jnp.int32),
        pltpu.VMEM((K, D), jnp.float32),
        pltpu.VMEM((K, D), jnp.float32),
        pltpu.VMEM((K, D), jnp.float32),
        pltpu.VMEM((K, D), jnp.float32),
        pltpu.VMEM((K,), jnp.int32),
        pltpu.VMEM((K,), jnp.int32),
        pltpu.SemaphoreType.DMA,
        pltpu.SemaphoreType.DMA,
        pltpu.SemaphoreType.DMA,
        pltpu.SemaphoreType.DMA,
        pltpu.SemaphoreType.DMA,
        pltpu.SemaphoreType.DMA,
    ],
)


def kernel(input_ids, token_type_ids, span_ids, word_emb, pos_emb, type_emb,
           span_emb, ln_gamma, ln_beta):
  wt_ids = (input_ids.astype(jnp.int32)
            + VOCAB * token_type_ids.astype(jnp.int32))
  ids3 = jnp.stack([
      wt_ids.reshape(N // K, K),
      span_ids.reshape(N // K, K).astype(jnp.int32),
  ], axis=1)
  wemb_aug = jnp.concatenate(
      [word_emb, word_emb + (type_emb[1] - type_emb[0])[None, :]], axis=0)
  pos_t0 = pos_emb[:S] + type_emb[0][None, :]
  out = _sc_embed_ln(ids3, wemb_aug, pos_t0, span_emb, ln_gamma, ln_beta)
  return out.reshape(B, S, D)


# speedup vs baseline: 1.0006x; 1.0006x over previous
import jax
import jax.numpy as jnp
from jax import lax
from jax.experimental import pallas as pl
from jax.experimental.pallas import tpu as pltpu
from jax.experimental.pallas import tpu_sc as plsc

B, S, D = 4096, 200, 128
N = B * S
VOCAB = 100000
NC, NS = 2, 16
NW = NC * NS
TOK_PER_W = N // NW
K = 64
CHUNKS = TOK_PER_W // K
EPS = 1e-12

_mesh = plsc.VectorSubcoreMesh(
    core_axis_name="c", subcore_axis_name="s", num_cores=NC, num_subcores=NS)


def _bcast16(vec, j):
  return vec.at[jnp.full((16,), j, jnp.int32)].get(mode="promise_in_bounds")


def _sc_embed_ln_body(ids3_hbm, wemb_hbm, pos_hbm, semb_hbm,
                      g_hbm, b_hbm, out_hbm,
                      pos_tab, span_tab, g_tab, b_tab,
                      ibuf0, ibuf1, wbuf0, wbuf1, obuf0, obuf1, svb0, svb1,
                      isem0, isem1, gsem0, gsem1, osem0, osem1):
  wid = lax.axis_index("s") * NC + lax.axis_index("c")
  wbase = wid * TOK_PER_W
  cbase = wid * CHUNKS

  pltpu.sync_copy(pos_hbm, pos_tab)
  pltpu.sync_copy(semb_hbm, span_tab)
  pltpu.sync_copy(g_hbm, g_tab)
  pltpu.sync_copy(b_hbm, b_tab)

  lanes = lax.iota(jnp.int32, 16)
  zero = jnp.zeros((16,), jnp.float32)

  def issue_ids(c, ibuf, isem):
    pltpu.async_copy(ids3_hbm.at[cbase + c], ibuf, isem)

  def wait_ids(c, ibuf, isem):
    pltpu.make_async_copy(ids3_hbm.at[cbase + c], ibuf, isem).wait()

  def issue_gather(ibuf, wbuf, gsem):
    pltpu.async_copy(wemb_hbm.at[ibuf.at[0]], wbuf, gsem)

  def wait_gather(ibuf, wbuf, gsem):
    pltpu.make_async_copy(wemb_hbm.at[ibuf.at[0]], wbuf, gsem).wait()

  def issue_out(c, obuf, osem):
    pltpu.async_copy(obuf, out_hbm.at[pl.ds(wbase + c * K, K)], osem)

  def wait_out(c, obuf, osem):
    pltpu.make_async_copy(
        obuf, out_hbm.at[pl.ds(wbase + c * K, K)], osem).wait()

  def stage_sv(ibuf, svb):
    for g in range(K // 16):
      svb[pl.ds(g * 16, 16)] = ibuf[1, pl.ds(g * 16, 16)]

  def compute(cloc, svb, wbuf, obuf):
    gstrips = [g_tab[pl.ds(kk * 16, 16)] for kk in range(D // 16)]
    bstrips = [b_tab[pl.ds(kk * 16, 16)] for kk in range(D // 16)]

    def group_body(g, carry2):
      lt = g * 16 + lanes
      pv = lax.rem(cloc * K + lt, S)
      sv = svb[pl.ds(g * 16, 16)]

      def pass1(d, accs):
        s_, q_ = accs
        dl = lax.bitwise_and(lanes + d, D - 1)
        x = (plsc.load_gather(wbuf, [lt, dl])
             + plsc.load_gather(span_tab, [sv, dl])
             + plsc.load_gather(pos_tab, [pv, dl]))
        plsc.store_scatter(wbuf, [lt, dl], x)
        return s_ + x, q_ + x * x

      s_, q_ = plsc.parallel_loop(
          0, D, unroll=8, carry=(zero, zero))(pass1)
      mean = s_ * (1.0 / D)
      var = q_ * (1.0 / D) - mean * mean
      v = var + EPS
      y = plsc.bitcast(
          jnp.int32(0x5F3759DF) - lax.shift_right_logical(
              plsc.bitcast(v, jnp.int32), 1), jnp.float32)
      y = y * (1.5 - 0.5 * v * y * y)
      y = y * (1.5 - 0.5 * v * y * y)
      y = y * (1.5 - 0.5 * v * y * y)
      shift = -mean * y

      @plsc.parallel_loop(0, 16, unroll=4)
      def _(t):
        tok = g * 16 + t
        yt = _bcast16(y, t)
        st = _bcast16(shift, t)
        for kk in range(D // 16):
          x = wbuf[tok, pl.ds(kk * 16, 16)]
          obuf[tok, pl.ds(kk * 16, 16)] = (
              (x * yt + st) * gstrips[kk] + bstrips[kk])

      return carry2

    lax.fori_loop(0, K // 16, group_body, 0)

  pltpu.sync_copy(ids3_hbm.at[cbase], ibuf0)
  issue_gather(ibuf0, wbuf0, gsem0)
  issue_ids(1, ibuf1, isem1)

  def step(i, carry):
    a = 2 * i
    b = a + 1
    wait_ids(b, ibuf1, isem1)
    issue_gather(ibuf1, wbuf1, gsem1)
    wait_gather(ibuf0, wbuf0, gsem0)
    stage_sv(ibuf0, svb0)

    @pl.when(a + 2 < CHUNKS)
    def _():
      issue_ids(a + 2, ibuf0, isem0)

    @pl.when(i > 0)
    def _():
      wait_out(a - 2, obuf0, osem0)

    compute(a, svb0, wbuf0, obuf0)
    issue_out(a, obuf0, osem0)

    @pl.when(a + 2 < CHUNKS)
    def _():
      wait_ids(a + 2, ibuf0, isem0)
      issue_gather(ibuf0, wbuf0, gsem0)

    wait_gather(ibuf1, wbuf1, gsem1)
    stage_sv(ibuf1, svb1)

    @pl.when(b + 2 < CHUNKS)
    def _():
      issue_ids(b + 2, ibuf1, isem1)

    @pl.when(i > 0)
    def _():
      wait_out(b - 2, obuf1, osem1)

    compute(b, svb1, wbuf1, obuf1)
    issue_out(b, obuf1, osem1)

    return carry

  lax.fori_loop(0, CHUNKS // 2, step, 0)
  wait_out(CHUNKS - 2, obuf0, osem0)
  wait_out(CHUNKS - 1, obuf1, osem1)


_sc_embed_ln = pl.kernel(
    _sc_embed_ln_body,
    out_type=jax.ShapeDtypeStruct((N, D), jnp.float32),
    mesh=_mesh,
    compiler_params=pltpu.CompilerParams(needs_layout_passes=False),
    scratch_types=[
        pltpu.VMEM((S, D), jnp.float32),
        pltpu.VMEM((512, D), jnp.float32),
        pltpu.VMEM((D,), jnp.float32),
        pltpu.VMEM((D,), jnp.float32),
        pltpu.VMEM((2, K), jnp.int32),
        pltpu.VMEM((2, K), jnp.int32),
        pltpu.VMEM((K, D), jnp.float32),
        pltpu.VMEM((K, D), jnp.float32),
        pltpu.VMEM((K, D), jnp.float32),
        pltpu.VMEM((K, D), jnp.float32),
        pltpu.VMEM((K,), jnp.int32),
        pltpu.VMEM((K,), jnp.int32),
        pltpu.SemaphoreType.DMA,
        pltpu.SemaphoreType.DMA,
        pltpu.SemaphoreType.DMA,
        pltpu.SemaphoreType.DMA,
        pltpu.SemaphoreType.DMA,
        pltpu.SemaphoreType.DMA,
    ],
)


def kernel(input_ids, token_type_ids, span_ids, word_emb, pos_emb, type_emb,
           span_emb, ln_gamma, ln_beta):
  wt_ids = (input_ids.astype(jnp.int32)
            + VOCAB * token_type_ids.astype(jnp.int32))
  ids3 = jnp.stack([
      wt_ids.reshape(N // K, K),
      span_ids.reshape(N // K, K).astype(jnp.int32),
  ], axis=1)
  wemb_aug = jnp.concatenate(
      [word_emb, word_emb + (type_emb[1] - type_emb[0])[None, :]], axis=0)
  pos_t0 = pos_emb[:S] + type_emb[0][None, :]
  out = _sc_embed_ln(ids3, wemb_aug, pos_t0, span_emb, ln_gamma, ln_beta)
  return out.reshape(B, S, D)
